# baseline (speedup 1.0000x reference)
import functools

import jax
import jax.numpy as jnp
from jax import lax
from jax.experimental import pallas as pl
from jax.experimental.pallas import tpu as pltpu
from jax.experimental.pallas import tpu_sc as plsc

L = 16
NC = 2
NS = 16
NW = NC * NS
KP = 112


def _hazard_body(t_hbm, s_hbm, tab_hbm, llo_hbm, ch_hbm,
                 t_v, s_v, tab_v, d_t, o1_v, o2_v,
                 sem_tab, sem_b, sem_o, *, chunk):
    wid = lax.axis_index("s") * NC + lax.axis_index("c")
    base = wid * chunk

    cp_tab = pltpu.async_copy(tab_hbm, tab_v, sem_tab)
    cp_t = pltpu.async_copy(t_hbm.at[pl.ds(base, chunk)], t_v, sem_b)
    cp_s = pltpu.async_copy(s_hbm.at[pl.ds(base, chunk)], s_v, sem_b)
    cp_tab.wait()

    ll_t = tab_v.at[pl.ds(0, KP)]

    lane = lax.iota(jnp.int32, L)
    dn = lax.GatherDimensionNumbers(
        offset_dims=(), collapsed_slice_dims=(0,), start_index_map=(0,))

    def lane_gather(x, idx):
        return lax.gather(x, idx[:, None], dn, slice_sizes=(1,),
                          mode=lax.GatherScatterMode.PROMISE_IN_BOUNDS)

    def table_step(c, carry):
        sl = pl.ds(c * L, L)
        lam = jnp.exp(tab_v[pl.ds(c * L, L)])
        prod = lam * tab_v[pl.ds(2 * KP + c * L, L)]
        incl = prod
        for d in (1, 2, 4, 8):
            shifted = lane_gather(incl, jnp.maximum(lane - d, 0))
            incl = incl + jnp.where(lane >= d, shifted, jnp.float32(0.0))
        d_t[sl] = (incl - prod) + carry - lam * tab_v[pl.ds(KP + c * L, L)]
        total = lane_gather(incl, jnp.full((L,), L - 1, jnp.int32))
        return carry + total

    lax.fori_loop(0, KP // L, table_step, jnp.zeros((L,), jnp.float32),
                  unroll=True)

    cp_t.wait()
    cp_s.wait()

    def batch_step(i, carry):
        sl = pl.ds(i * L, L)
        s = s_v[sl]
        tt = t_v[sl]
        llg = plsc.load_gather(ll_t, [s])
        o1_v[sl] = llg
        o2_v[sl] = jnp.exp(llg) * tt + plsc.load_gather(d_t, [s])
        return carry

    lax.fori_loop(0, chunk // L, batch_step, 0, unroll=True)

    c1 = pltpu.async_copy(o1_v, llo_hbm.at[pl.ds(base, chunk)], sem_o)
    c2 = pltpu.async_copy(o2_v, ch_hbm.at[pl.ds(base, chunk)], sem_o)
    c1.wait()
    c2.wait()


@jax.jit
def kernel(t, t_section, log_lambda, breakpoints, widths):
    b = t.shape[0]
    k = log_lambda.shape[0]
    chunk = b // NW

    pad = KP - k
    tab = jnp.concatenate([
        jnp.pad(log_lambda[:, 0], (0, pad)),
        jnp.pad(breakpoints, (0, pad)),
        jnp.pad(widths[:, 0], (0, pad)),
    ])
    t_flat = t[:, 0]
    s = t_section.astype(jnp.int32)

    f32 = jnp.float32
    run = pl.kernel(
        functools.partial(_hazard_body, chunk=chunk),
        out_type=(
            jax.ShapeDtypeStruct((b,), f32),
            jax.ShapeDtypeStruct((b,), f32),
        ),
        mesh=plsc.VectorSubcoreMesh(
            core_axis_name="c", subcore_axis_name="s",
            num_cores=NC, num_subcores=NS,
        ),
        compiler_params=pltpu.CompilerParams(needs_layout_passes=False),
        scratch_types=[
            pltpu.VMEM((chunk,), f32),
            pltpu.VMEM((chunk,), jnp.int32),
            pltpu.VMEM((3 * KP,), f32),
            pltpu.VMEM((KP,), f32),
            pltpu.VMEM((chunk,), f32),
            pltpu.VMEM((chunk,), f32),
            pltpu.SemaphoreType.DMA,
            pltpu.SemaphoreType.DMA,
            pltpu.SemaphoreType.DMA,
        ],
    )
    llo, ch = run(t_flat, s, tab)
    return llo[:, None], ch[:, None]


# speedup vs baseline: 1.0280x; 1.0280x over previous
import functools

import jax
import jax.numpy as jnp
from jax import lax
from jax.experimental import pallas as pl
from jax.experimental.pallas import tpu as pltpu
from jax.experimental.pallas import tpu_sc as plsc

L = 16
NC = 2
NS = 16
NW = NC * NS
KP = 112


def _hazard_body(t_hbm, s_hbm, tab_hbm, llo_hbm, ch_hbm,
                 t_v, s_v, tab_v, lam_t, d_t, o1_v, o2_v,
                 sem_tab, sem_b, sem_o, *, chunk):
    wid = lax.axis_index("s") * NC + lax.axis_index("c")
    base = wid * chunk

    cp_tab = pltpu.async_copy(tab_hbm, tab_v, sem_tab)
    cp_t = pltpu.async_copy(t_hbm.at[pl.ds(base, chunk)], t_v, sem_b)
    cp_s = pltpu.async_copy(s_hbm.at[pl.ds(base, chunk)], s_v, sem_b)
    cp_tab.wait()

    ll_t = tab_v.at[pl.ds(0, KP)]

    lane = lax.iota(jnp.int32, L)
    dn = lax.GatherDimensionNumbers(
        offset_dims=(), collapsed_slice_dims=(0,), start_index_map=(0,))

    def lane_gather(x, idx):
        return lax.gather(x, idx[:, None], dn, slice_sizes=(1,),
                          mode=lax.GatherScatterMode.PROMISE_IN_BOUNDS)

    def table_step(c, carry):
        sl = pl.ds(c * L, L)
        lam = jnp.exp(tab_v[pl.ds(c * L, L)])
        prod = lam * tab_v[pl.ds(2 * KP + c * L, L)]
        incl = prod
        for d in (1, 2, 4, 8):
            shifted = lane_gather(incl, jnp.maximum(lane - d, 0))
            incl = incl + jnp.where(lane >= d, shifted, jnp.float32(0.0))
        lam_t[sl] = lam
        d_t[sl] = (incl - prod) + carry - lam * tab_v[pl.ds(KP + c * L, L)]
        total = lane_gather(incl, jnp.full((L,), L - 1, jnp.int32))
        return carry + total

    lax.fori_loop(0, KP // L, table_step, jnp.zeros((L,), jnp.float32),
                  unroll=True)

    cp_t.wait()
    cp_s.wait()

    def batch_step(i, carry):
        sl = pl.ds(i * L, L)
        s = s_v[sl]
        tt = t_v[sl]
        o1_v[sl] = plsc.load_gather(ll_t, [s])
        o2_v[sl] = plsc.load_gather(lam_t, [s]) * tt + plsc.load_gather(d_t, [s])
        return carry

    lax.fori_loop(0, chunk // L, batch_step, 0, unroll=4)

    c1 = pltpu.async_copy(o1_v, llo_hbm.at[pl.ds(base, chunk)], sem_o)
    c2 = pltpu.async_copy(o2_v, ch_hbm.at[pl.ds(base, chunk)], sem_o)
    c1.wait()
    c2.wait()


@jax.jit
def kernel(t, t_section, log_lambda, breakpoints, widths):
    b = t.shape[0]
    k = log_lambda.shape[0]
    chunk = b // NW

    pad = KP - k
    tab = jnp.concatenate([
        jnp.pad(log_lambda[:, 0], (0, pad)),
        jnp.pad(breakpoints, (0, pad)),
        jnp.pad(widths[:, 0], (0, pad)),
    ])
    t_flat = t[:, 0]
    s = t_section.astype(jnp.int32)

    f32 = jnp.float32
    run = pl.kernel(
        functools.partial(_hazard_body, chunk=chunk),
        out_type=(
            jax.ShapeDtypeStruct((b,), f32),
            jax.ShapeDtypeStruct((b,), f32),
        ),
        mesh=plsc.VectorSubcoreMesh(
            core_axis_name="c", subcore_axis_name="s",
            num_cores=NC, num_subcores=NS,
        ),
        compiler_params=pltpu.CompilerParams(needs_layout_passes=False),
        scratch_types=[
            pltpu.VMEM((chunk,), f32),
            pltpu.VMEM((chunk,), jnp.int32),
            pltpu.VMEM((3 * KP,), f32),
            pltpu.VMEM((KP,), f32),
            pltpu.VMEM((KP,), f32),
            pltpu.VMEM((chunk,), f32),
            pltpu.VMEM((chunk,), f32),
            pltpu.SemaphoreType.DMA,
            pltpu.SemaphoreType.DMA,
            pltpu.SemaphoreType.DMA,
        ],
    )
    llo, ch = run(t_flat, s, tab)
    return llo[:, None], ch[:, None]


# speedup vs baseline: 1.0494x; 1.0207x over previous
import functools

import jax
import jax.numpy as jnp
from jax import lax
from jax.experimental import pallas as pl
from jax.experimental.pallas import tpu as pltpu
from jax.experimental.pallas import tpu_sc as plsc

L = 16
NC = 2
NS = 16
NW = NC * NS
KP = 112


def _hazard_body(t_hbm, s_hbm, tab_hbm, llo_hbm, ch_hbm,
                 t_v, s_v, tab_v, lam_t, d_t, o1_v, o2_v,
                 sem_tab, sem_b, sem_o, *, chunk):
    wid = lax.axis_index("s") * NC + lax.axis_index("c")
    base = wid * chunk

    cp_tab = pltpu.async_copy(tab_hbm, tab_v, sem_tab)
    cp_t = pltpu.async_copy(t_hbm.at[pl.ds(base, chunk)], t_v, sem_b)
    cp_s = pltpu.async_copy(s_hbm.at[pl.ds(base, chunk)], s_v, sem_b)
    cp_tab.wait()

    ll_t = tab_v.at[pl.ds(0, KP)]

    lane = lax.iota(jnp.int32, L)
    dn = lax.GatherDimensionNumbers(
        offset_dims=(), collapsed_slice_dims=(0,), start_index_map=(0,))

    def lane_gather(x, idx):
        return lax.gather(x, idx[:, None], dn, slice_sizes=(1,),
                          mode=lax.GatherScatterMode.PROMISE_IN_BOUNDS)

    def table_step(c, carry):
        sl = pl.ds(c * L, L)
        lam = jnp.exp(tab_v[pl.ds(c * L, L)])
        prod = lam * tab_v[pl.ds(2 * KP + c * L, L)]
        incl = prod
        for d in (1, 2, 4, 8):
            shifted = lane_gather(incl, jnp.maximum(lane - d, 0))
            incl = incl + jnp.where(lane >= d, shifted, jnp.float32(0.0))
        lam_t[sl] = lam
        d_t[sl] = (incl - prod) + carry - lam * tab_v[pl.ds(KP + c * L, L)]
        total = lane_gather(incl, jnp.full((L,), L - 1, jnp.int32))
        return carry + total

    lax.fori_loop(0, KP // L, table_step, jnp.zeros((L,), jnp.float32),
                  unroll=True)

    cp_t.wait()
    cp_s.wait()

    @plsc.parallel_loop(0, chunk // L, unroll=4)
    def batch_step(i):
        sl = pl.ds(i * L, L)
        s = s_v[sl]
        tt = t_v[sl]
        o1_v[sl] = plsc.load_gather(ll_t, [s])
        o2_v[sl] = plsc.load_gather(lam_t, [s]) * tt + plsc.load_gather(d_t, [s])

    c1 = pltpu.async_copy(o1_v, llo_hbm.at[pl.ds(base, chunk)], sem_o)
    c2 = pltpu.async_copy(o2_v, ch_hbm.at[pl.ds(base, chunk)], sem_o)
    c1.wait()
    c2.wait()


@jax.jit
def kernel(t, t_section, log_lambda, breakpoints, widths):
    b = t.shape[0]
    k = log_lambda.shape[0]
    chunk = b // NW

    pad = KP - k
    tab = jnp.concatenate([
        jnp.pad(log_lambda[:, 0], (0, pad)),
        jnp.pad(breakpoints, (0, pad)),
        jnp.pad(widths[:, 0], (0, pad)),
    ])
    t_flat = t[:, 0]
    s = t_section.astype(jnp.int32)

    f32 = jnp.float32
    run = pl.kernel(
        functools.partial(_hazard_body, chunk=chunk),
        out_type=(
            jax.ShapeDtypeStruct((b,), f32),
            jax.ShapeDtypeStruct((b,), f32),
        ),
        mesh=plsc.VectorSubcoreMesh(
            core_axis_name="c", subcore_axis_name="s",
            num_cores=NC, num_subcores=NS,
        ),
        compiler_params=pltpu.CompilerParams(needs_layout_passes=False),
        scratch_types=[
            pltpu.VMEM((chunk,), f32),
            pltpu.VMEM((chunk,), jnp.int32),
            pltpu.VMEM((3 * KP,), f32),
            pltpu.VMEM((KP,), f32),
            pltpu.VMEM((KP,), f32),
            pltpu.VMEM((chunk,), f32),
            pltpu.VMEM((chunk,), f32),
            pltpu.SemaphoreType.DMA,
            pltpu.SemaphoreType.DMA,
            pltpu.SemaphoreType.DMA,
        ],
    )
    llo, ch = run(t_flat, s, tab)
    return llo[:, None], ch[:, None]
